# baseline (speedup 1.0000x reference)
import jax
import jax.numpy as jnp
from jax.experimental import pallas as pl
from jax.experimental.pallas import tpu as pltpu

H, DN, DV, DD, BLK, RBX = 128, 128, 128, 64, 256, 512
DIM, RC = 7168, 1536
S = 2048

HB = 16
NHB = H // HB
NC = S // BLK
PROJ_BS = 512
PROJ_KT = 1792
NKB = DIM // PROJ_KT
SUB = 256
NS = BLK // SUB

_F32 = jnp.float32
_BF16 = jnp.bfloat16


def _proj_kernel(x_ref, wca_ref, wbxa_ref, xa_ref, bx_ref, acc_a, acc_b):
    k = pl.program_id(1)
    dn = (((1,), (1,)), ((), ()))
    xb = x_ref[...].astype(_BF16)
    pa = jax.lax.dot_general(xb, wca_ref[...], dn,
                             preferred_element_type=_F32)
    pb = jax.lax.dot_general(xb, wbxa_ref[...], dn,
                             preferred_element_type=_F32)

    @pl.when(k == 0)
    def _():
        acc_a[...] = pa
        acc_b[...] = pb

    @pl.when(k != 0)
    def _():
        acc_a[...] += pa
        acc_b[...] += pb

    @pl.when(k == NKB - 1)
    def _():
        xa_ref[...] = acc_a[...].astype(_BF16)
        bx_ref[...] = acc_b[...]


def _ssd_kernel(xa_ref, bx_ref, wc_ref, wdt_ref, wb_ref, wx_ref, a_ref,
                e_ref, tbd_ref, tf_ref, out_ref, state_ref):
    c = pl.program_id(1)

    @pl.when(c == 0)
    def _():
        state_ref[...] = jnp.zeros_like(state_ref)

    xa = xa_ref[...]
    bxl = bx_ref[:, :RBX].astype(_BF16)
    dtb = bx_ref[:, RBX:]

    dn_tb = (((1,), (1,)), ((), ()))
    cC = jax.lax.dot_general(xa, wc_ref[...], dn_tb,
                             preferred_element_type=_F32).astype(_BF16)
    cD = jax.lax.dot_general(xa, wdt_ref[...], dn_tb,
                             preferred_element_type=_F32)
    bB = jax.lax.dot_general(bxl, wb_ref[...], dn_tb,
                             preferred_element_type=_F32).astype(_BF16)
    bX = jax.lax.dot_general(bxl, wx_ref[...], dn_tb,
                             preferred_element_type=_F32).astype(_BF16)

    tril_bd = tbd_ref[...]
    tril_f = tf_ref[...]

    dt_cols = [
        jnp.sum(cD[:, h * DD:(h + 1) * DD] * dtb, axis=1, keepdims=True)
        for h in range(HB)
    ]
    dt_pre = jnp.concatenate(dt_cols, axis=1)
    dt_all = jnp.maximum(dt_pre, 0.0) + jnp.log1p(jnp.exp(-jnp.abs(dt_pre)))
    dA_all = dt_all * a_ref[0]
    acs_all = jax.lax.dot_general(
        tril_bd, dA_all.astype(_BF16), (((1,), (0,)), ((), ())),
        preferred_element_type=_F32)
    acs_t = acs_all.T
    exp_acs = jnp.exp(acs_all)
    ends = [acs_all[s * SUB + SUB - 1:s * SUB + SUB, :] for s in range(NS)]
    ds_all = jnp.concatenate(
        [jnp.exp(ends[s] - acs_all[s * SUB:(s + 1) * SUB, :])
         for s in range(NS)], axis=0)
    exp_tot = [jnp.exp(e) for e in ends]

    e_mat = e_ref[...]
    dtexp = jnp.dot(dt_all.astype(_BF16), e_mat,
                    preferred_element_type=_F32).astype(_BF16)
    dsexp = jnp.dot(ds_all.astype(_BF16), e_mat,
                    preferred_element_type=_F32).astype(_BF16)
    Xd_all = bX * dtexp
    Xds_all = Xd_all * dsexp

    dn_ta = (((0,), (0,)), ((), ()))
    for h in range(HB):
        st = state_ref[h]
        for s in range(NS):
            r0, r1 = s * SUB, (s + 1) * SUB
            Cm = cC[r0:r1, h * DN:(h + 1) * DN]
            Bm = bB[r0:r1, h * DN:(h + 1) * DN]
            Xd = Xd_all[r0:r1, h * DV:(h + 1) * DV]
            Xds = Xds_all[r0:r1, h * DV:(h + 1) * DV]

            seg = acs_all[r0:r1, h:h + 1] - acs_t[h:h + 1, r0:r1]
            Lm = jnp.exp(jnp.minimum(seg, 0.0)) * tril_f

            scores = jax.lax.dot_general(Cm, Bm, dn_tb,
                                         preferred_element_type=_F32) * Lm
            Yd = jnp.dot(scores.astype(_BF16), Xd,
                         preferred_element_type=_F32)

            states_c = jax.lax.dot_general(Bm, Xds, dn_ta,
                                           preferred_element_type=_F32)

            Yoff = jnp.dot(Cm, st.astype(_BF16),
                           preferred_element_type=_F32) \
                * exp_acs[r0:r1, h:h + 1]
            st = st * exp_tot[s][:, h:h + 1] + states_c
            out_ref[r0:r1, h, :] = Yd + Yoff
        state_ref[h] = st


@jax.jit
def _lsm_fused(x2, wc_a, w_c, w_dt, wbx_a, w_b, w_x, a_neg, e_mat, t_bd, t_f):
    xa, bx = pl.pallas_call(
        _proj_kernel,
        grid=(S // PROJ_BS, NKB),
        in_specs=[
            pl.BlockSpec((PROJ_BS, PROJ_KT), lambda i, k: (i, k)),
            pl.BlockSpec((RC, PROJ_KT), lambda i, k: (0, k)),
            pl.BlockSpec((RBX + DD, PROJ_KT), lambda i, k: (0, k)),
        ],
        out_specs=[
            pl.BlockSpec((PROJ_BS, RC), lambda i, k: (i, 0)),
            pl.BlockSpec((PROJ_BS, RBX + DD), lambda i, k: (i, 0)),
        ],
        out_shape=[
            jax.ShapeDtypeStruct((S, RC), _BF16),
            jax.ShapeDtypeStruct((S, RBX + DD), _F32),
        ],
        scratch_shapes=[
            pltpu.VMEM((PROJ_BS, RC), _F32),
            pltpu.VMEM((PROJ_BS, RBX + DD), _F32),
        ],
        compiler_params=pltpu.CompilerParams(
            dimension_semantics=("parallel", "arbitrary"),
            vmem_limit_bytes=52 * 1024 * 1024,
        ),
        name="lsm_proj",
    )(x2, wc_a, wbx_a)

    y = pl.pallas_call(
        _ssd_kernel,
        grid=(NHB, NC),
        in_specs=[
            pl.BlockSpec((BLK, RC), lambda hb, c: (c, 0)),
            pl.BlockSpec((BLK, RBX + DD), lambda hb, c: (c, 0)),
            pl.BlockSpec((HB * DN, RC), lambda hb, c: (hb, 0)),
            pl.BlockSpec((HB * DD, RC), lambda hb, c: (hb, 0)),
            pl.BlockSpec((HB * DN, RBX), lambda hb, c: (hb, 0)),
            pl.BlockSpec((HB * DV, RBX), lambda hb, c: (hb, 0)),
            pl.BlockSpec((1, 1, HB), lambda hb, c: (hb, 0, 0)),
            pl.BlockSpec((HB, HB * DV), lambda hb, c: (0, 0)),
            pl.BlockSpec((BLK, BLK), lambda hb, c: (0, 0)),
            pl.BlockSpec((SUB, SUB), lambda hb, c: (0, 0)),
        ],
        out_specs=pl.BlockSpec((BLK, HB, DV), lambda hb, c: (c, hb, 0)),
        out_shape=jax.ShapeDtypeStruct((S, H, DV), _F32),
        scratch_shapes=[pltpu.VMEM((HB, DN, DV), _F32)],
        compiler_params=pltpu.CompilerParams(
            dimension_semantics=("parallel", "arbitrary"),
            vmem_limit_bytes=52 * 1024 * 1024,
        ),
        name="lsm_ssd",
    )(xa, bx, w_c, w_dt, w_b, w_x, a_neg, e_mat, t_bd, t_f)
    return y


def kernel(x, wc_a, wc_b, wbx_a, wbx_b, A_log):
    b, s, _ = x.shape
    x2 = x.reshape(s, x.shape[-1])
    a_neg = (-jnp.exp(A_log)).reshape(NHB, 1, HB)
    wcb3 = wc_b.reshape(H, DN + DD, RC)
    w_c = wcb3[:, :DN, :].reshape(H * DN, RC).astype(_BF16)
    w_dt = wcb3[:, DN:, :].reshape(H * DD, RC).astype(_BF16)
    wbx3 = wbx_b.reshape(H, DN + DV, RBX)
    w_b = wbx3[:, :DN, :].reshape(H * DN, RBX).astype(_BF16)
    w_x = wbx3[:, DN:, :].reshape(H * DV, RBX).astype(_BF16)
    lane_head = jnp.arange(HB * DV, dtype=jnp.int32)[None, :] // DV
    e_mat = (lane_head == jnp.arange(HB, dtype=jnp.int32)[:, None]) \
        .astype(_BF16)
    ib = jnp.arange(BLK, dtype=jnp.int32)
    t_bd = ((ib[:, None] >= ib[None, :])
            & (ib[:, None] // SUB == ib[None, :] // SUB)).astype(_BF16)
    isub = jnp.arange(SUB, dtype=jnp.int32)
    t_f = (isub[:, None] >= isub[None, :]).astype(_F32)
    y = _lsm_fused(x2, wc_a.astype(_BF16), w_c, w_dt,
                   wbx_a.astype(_BF16), w_b, w_x, a_neg, e_mat, t_bd, t_f)
    return y.reshape(b, s, H, DV)


# speedup vs baseline: 1.0075x; 1.0075x over previous
import jax
import jax.numpy as jnp
from jax.experimental import pallas as pl
from jax.experimental.pallas import tpu as pltpu

H, DN, DV, DD, BLK, RBX = 128, 128, 128, 64, 256, 512
DIM, RC = 7168, 1536
S = 2048

HB = 16
NHB = H // HB
NC = S // BLK
PROJ_BS = 512
PROJ_KT = 1792
NKB = DIM // PROJ_KT
SUB = 256
NS = BLK // SUB

_F32 = jnp.float32
_BF16 = jnp.bfloat16


def _proj_kernel(x_ref, wca_ref, wbxa_ref, xa_ref, bx_ref, acc_a, acc_b):
    k = pl.program_id(1)
    dn = (((1,), (1,)), ((), ()))
    xb = x_ref[...].astype(_BF16)
    pa = jax.lax.dot_general(xb, wca_ref[...], dn,
                             preferred_element_type=_F32)
    pb = jax.lax.dot_general(xb, wbxa_ref[...], dn,
                             preferred_element_type=_F32)

    @pl.when(k == 0)
    def _():
        acc_a[...] = pa
        acc_b[...] = pb

    @pl.when(k != 0)
    def _():
        acc_a[...] += pa
        acc_b[...] += pb

    @pl.when(k == NKB - 1)
    def _():
        xa_ref[...] = acc_a[...].astype(_BF16)
        bx_ref[...] = acc_b[...]


def _ssd_kernel(xa_ref, bx_ref, wc_ref, wdt_ref, wb_ref, wx_ref, a_ref,
                e_ref, out_ref, state_ref):
    c = pl.program_id(1)

    @pl.when(c == 0)
    def _():
        state_ref[...] = jnp.zeros_like(state_ref)

    xa = xa_ref[...]
    bxl = bx_ref[:, :RBX].astype(_BF16)
    dtb = bx_ref[:, RBX:]

    dn_tb = (((1,), (1,)), ((), ()))
    cC = jax.lax.dot_general(xa, wc_ref[...], dn_tb,
                             preferred_element_type=_F32).astype(_BF16)
    cD = jax.lax.dot_general(xa, wdt_ref[...], dn_tb,
                             preferred_element_type=_F32)
    bB = jax.lax.dot_general(bxl, wb_ref[...], dn_tb,
                             preferred_element_type=_F32).astype(_BF16)
    bX = jax.lax.dot_general(bxl, wx_ref[...], dn_tb,
                             preferred_element_type=_F32).astype(_BF16)

    ib = jax.lax.broadcasted_iota(jnp.int32, (BLK, BLK), 0)
    jb = jax.lax.broadcasted_iota(jnp.int32, (BLK, BLK), 1)
    tril_bd = jnp.where((ib >= jb) & (ib // SUB == jb // SUB), 1.0, 0.0) \
        .astype(_BF16)
    ii = jax.lax.broadcasted_iota(jnp.int32, (SUB, SUB), 0)
    jj = jax.lax.broadcasted_iota(jnp.int32, (SUB, SUB), 1)
    tril = ii >= jj

    dt_cols = [
        jnp.sum(cD[:, h * DD:(h + 1) * DD] * dtb, axis=1, keepdims=True)
        for h in range(HB)
    ]
    dt_pre = jnp.concatenate(dt_cols, axis=1)
    dt_all = jnp.maximum(dt_pre, 0.0) + jnp.log1p(jnp.exp(-jnp.abs(dt_pre)))
    dA_all = dt_all * a_ref[0]
    acs_all = jax.lax.dot_general(
        tril_bd, dA_all.astype(_BF16), (((1,), (0,)), ((), ())),
        preferred_element_type=_F32)
    acs_t = acs_all.T
    exp_acs = jnp.exp(acs_all)
    ends = [acs_all[s * SUB + SUB - 1:s * SUB + SUB, :] for s in range(NS)]
    ds_all = jnp.concatenate(
        [jnp.exp(ends[s] - acs_all[s * SUB:(s + 1) * SUB, :])
         for s in range(NS)], axis=0)
    exp_tot = [jnp.exp(e) for e in ends]

    e_mat = e_ref[...]
    dtexp = jnp.dot(dt_all.astype(_BF16), e_mat,
                    preferred_element_type=_F32).astype(_BF16)
    dsexp = jnp.dot(ds_all.astype(_BF16), e_mat,
                    preferred_element_type=_F32).astype(_BF16)
    Xd_all = bX * dtexp
    Xds_all = Xd_all * dsexp

    dn_ta = (((0,), (0,)), ((), ()))
    for h in range(HB):
        st = state_ref[h]
        for s in range(NS):
            r0, r1 = s * SUB, (s + 1) * SUB
            Cm = cC[r0:r1, h * DN:(h + 1) * DN]
            Bm = bB[r0:r1, h * DN:(h + 1) * DN]
            Xd = Xd_all[r0:r1, h * DV:(h + 1) * DV]
            Xds = Xds_all[r0:r1, h * DV:(h + 1) * DV]

            seg = acs_all[r0:r1, h:h + 1] - acs_t[h:h + 1, r0:r1]
            Lm = jnp.where(tril, jnp.exp(jnp.minimum(seg, 0.0)), 0.0)

            scores = jax.lax.dot_general(Cm, Bm, dn_tb,
                                         preferred_element_type=_F32) * Lm
            Yd = jnp.dot(scores.astype(_BF16), Xd,
                         preferred_element_type=_F32)

            states_c = jax.lax.dot_general(Bm, Xds, dn_ta,
                                           preferred_element_type=_F32)

            Yoff = jnp.dot(Cm, st.astype(_BF16),
                           preferred_element_type=_F32) \
                * exp_acs[r0:r1, h:h + 1]
            st = st * exp_tot[s][:, h:h + 1] + states_c
            out_ref[r0:r1, h, :] = Yd + Yoff
        state_ref[h] = st


@jax.jit
def _lsm_fused(x2, wc_a, w_c, w_dt, wbx_a, w_b, w_x, a_neg, e_mat):
    xa, bx = pl.pallas_call(
        _proj_kernel,
        grid=(S // PROJ_BS, NKB),
        in_specs=[
            pl.BlockSpec((PROJ_BS, PROJ_KT), lambda i, k: (i, k)),
            pl.BlockSpec((RC, PROJ_KT), lambda i, k: (0, k)),
            pl.BlockSpec((RBX + DD, PROJ_KT), lambda i, k: (0, k)),
        ],
        out_specs=[
            pl.BlockSpec((PROJ_BS, RC), lambda i, k: (i, 0)),
            pl.BlockSpec((PROJ_BS, RBX + DD), lambda i, k: (i, 0)),
        ],
        out_shape=[
            jax.ShapeDtypeStruct((S, RC), _BF16),
            jax.ShapeDtypeStruct((S, RBX + DD), _F32),
        ],
        scratch_shapes=[
            pltpu.VMEM((PROJ_BS, RC), _F32),
            pltpu.VMEM((PROJ_BS, RBX + DD), _F32),
        ],
        compiler_params=pltpu.CompilerParams(
            dimension_semantics=("parallel", "arbitrary"),
            vmem_limit_bytes=52 * 1024 * 1024,
        ),
        name="lsm_proj",
    )(x2, wc_a, wbx_a)

    y = pl.pallas_call(
        _ssd_kernel,
        grid=(NHB, NC),
        in_specs=[
            pl.BlockSpec((BLK, RC), lambda hb, c: (c, 0)),
            pl.BlockSpec((BLK, RBX + DD), lambda hb, c: (c, 0)),
            pl.BlockSpec((HB * DN, RC), lambda hb, c: (hb, 0)),
            pl.BlockSpec((HB * DD, RC), lambda hb, c: (hb, 0)),
            pl.BlockSpec((HB * DN, RBX), lambda hb, c: (hb, 0)),
            pl.BlockSpec((HB * DV, RBX), lambda hb, c: (hb, 0)),
            pl.BlockSpec((1, 1, HB), lambda hb, c: (hb, 0, 0)),
            pl.BlockSpec((HB, HB * DV), lambda hb, c: (0, 0)),
        ],
        out_specs=pl.BlockSpec((BLK, HB, DV), lambda hb, c: (c, hb, 0)),
        out_shape=jax.ShapeDtypeStruct((S, H, DV), _F32),
        scratch_shapes=[pltpu.VMEM((HB, DN, DV), _F32)],
        compiler_params=pltpu.CompilerParams(
            dimension_semantics=("parallel", "arbitrary"),
            vmem_limit_bytes=52 * 1024 * 1024,
        ),
        name="lsm_ssd",
    )(xa, bx, w_c, w_dt, w_b, w_x, a_neg, e_mat)
    return y


def kernel(x, wc_a, wc_b, wbx_a, wbx_b, A_log):
    b, s, _ = x.shape
    x2 = x.reshape(s, x.shape[-1])
    a_neg = (-jnp.exp(A_log)).reshape(NHB, 1, HB)
    wcb3 = wc_b.reshape(H, DN + DD, RC)
    w_c = wcb3[:, :DN, :].reshape(H * DN, RC).astype(_BF16)
    w_dt = wcb3[:, DN:, :].reshape(H * DD, RC).astype(_BF16)
    wbx3 = wbx_b.reshape(H, DN + DV, RBX)
    w_b = wbx3[:, :DN, :].reshape(H * DN, RBX).astype(_BF16)
    w_x = wbx3[:, DN:, :].reshape(H * DV, RBX).astype(_BF16)
    lane_head = jnp.arange(HB * DV, dtype=jnp.int32)[None, :] // DV
    e_mat = (lane_head == jnp.arange(HB, dtype=jnp.int32)[:, None]) \
        .astype(_BF16)
    y = _lsm_fused(x2, wc_a.astype(_BF16), w_c, w_dt,
                   wbx_a.astype(_BF16), w_b, w_x, a_neg, e_mat)
    return y.reshape(b, s, H, DV)
